# initial kernel scaffold (unmeasured)
import jax
import jax.numpy as jnp
from jax import lax
from jax.experimental import pallas as pl
from jax.experimental.pallas import tpu as pltpu


def kernel(
    x,
):
    def body(*refs):
        pass

    out_shape = jax.ShapeDtypeStruct(..., jnp.float32)
    return pl.pallas_call(body, out_shape=out_shape)(...)


# baseline (device time: 931272 ns/iter reference)
import jax
import jax.numpy as jnp
from jax import lax
from jax.experimental import pallas as pl
from jax.experimental.pallas import tpu as pltpu

M = 32768
N = 1024
CHUNK = 2048
N_CHUNKS = M // CHUNK


def kernel(x):
    x_bf = x.astype(jnp.bfloat16)

    def body(x_ref, out_ref, recv_ref, send_sem, recv_sem,
             a_ref, b_ref, o_ref, c_sems):
        my_x = lax.axis_index("x")
        my_y = lax.axis_index("y")
        my_z = lax.axis_index("z")
        partner = (1 - my_x, my_y, my_z)

        rdma = pltpu.make_async_remote_copy(
            src_ref=x_ref,
            dst_ref=recv_ref,
            send_sem=send_sem,
            recv_sem=recv_sem,
            device_id=partner,
            device_id_type=pl.DeviceIdType.MESH,
        )
        rdma.start()
        rdma.wait()

        for c in range(N_CHUNKS):
            sl = pl.ds(c * CHUNK, CHUNK)
            cp_a = pltpu.make_async_copy(x_ref.at[sl], a_ref, c_sems.at[0])
            cp_b = pltpu.make_async_copy(recv_ref.at[sl], b_ref, c_sems.at[1])
            cp_a.start()
            cp_b.start()
            cp_a.wait()
            cp_b.wait()
            o_ref[...] = a_ref[...] + b_ref[...]
            cp_o = pltpu.make_async_copy(o_ref, out_ref.at[sl], c_sems.at[2])
            cp_o.start()
            cp_o.wait()

    out, _recv = pl.pallas_call(
        body,
        out_shape=(
            jax.ShapeDtypeStruct((M, N), jnp.bfloat16),
            jax.ShapeDtypeStruct((M, N), jnp.bfloat16),
        ),
        in_specs=[pl.BlockSpec(memory_space=pltpu.MemorySpace.HBM)],
        out_specs=(
            pl.BlockSpec(memory_space=pltpu.MemorySpace.HBM),
            pl.BlockSpec(memory_space=pltpu.MemorySpace.HBM),
        ),
        scratch_shapes=[
            pltpu.SemaphoreType.DMA,
            pltpu.SemaphoreType.DMA,
            pltpu.VMEM((CHUNK, N), jnp.bfloat16),
            pltpu.VMEM((CHUNK, N), jnp.bfloat16),
            pltpu.VMEM((CHUNK, N), jnp.bfloat16),
            pltpu.SemaphoreType.DMA((3,)),
        ],
    )(x_bf)
    return out


# device time: 435275 ns/iter; 2.1395x vs baseline; 2.1395x over previous
import jax
import jax.numpy as jnp
from jax import lax
from jax.experimental import pallas as pl
from jax.experimental.pallas import tpu as pltpu

M = 32768
N = 1024
HALF = M // 2
CH = 1024
NC = HALF // CH
DEPTH = 4


def kernel(x):
    def body(x_ref, out_ref,
             fbuf, kbuf, sbuf, zbuf, recv1, recv2,
             ldf_sems, ldk_sems, st1_sems, st2_sems,
             send1_sems, send2_sems, recv1_sems, recv2_sems,
             credit1_sem, credit2_sem):
        my_x = lax.axis_index("x")
        my_y = lax.axis_index("y")
        my_z = lax.axis_index("z")
        p = my_z % 2
        K = my_x ^ p
        xp = (1 - my_x, my_y, my_z)
        zp = (my_x, my_y, my_z ^ 1)
        k_base = K * HALF
        s_base = (1 - K) * HALF

        def x_rdma(c):
            return pltpu.make_async_remote_copy(
                src_ref=sbuf.at[c % 2],
                dst_ref=recv1.at[c % DEPTH],
                send_sem=send1_sems.at[c % 2],
                recv_sem=recv1_sems.at[c % DEPTH],
                device_id=xp,
                device_id_type=pl.DeviceIdType.MESH,
            )

        def z_rdma(c):
            return pltpu.make_async_remote_copy(
                src_ref=zbuf.at[c % 2],
                dst_ref=recv2.at[c % DEPTH],
                send_sem=send2_sems.at[c % 2],
                recv_sem=recv2_sems.at[c % DEPTH],
                device_id=zp,
                device_id_type=pl.DeviceIdType.MESH,
            )

        def load_f(c):
            return pltpu.make_async_copy(
                x_ref.at[pl.ds(s_base + c * CH, CH)],
                fbuf.at[c % 2], ldf_sems.at[c % 2])

        def load_k(c):
            return pltpu.make_async_copy(
                x_ref.at[pl.ds(k_base + c * CH, CH)],
                kbuf.at[c % 2], ldk_sems.at[c % 2])

        def store1(c):
            return pltpu.make_async_copy(
                zbuf.at[c % 2],
                out_ref.at[pl.ds(k_base + c * CH, CH)], st1_sems.at[c % 2])

        def store2(c):
            return pltpu.make_async_copy(
                recv2.at[c % DEPTH],
                out_ref.at[pl.ds(s_base + c * CH, CH)], st2_sems.at[c % 2])

        barrier_sem = pltpu.get_barrier_semaphore()
        for nbr in (xp, zp):
            pl.semaphore_signal(barrier_sem, inc=1, device_id=nbr,
                                device_id_type=pl.DeviceIdType.MESH)
        pl.semaphore_wait(barrier_sem, 2)

        for t in range(NC + 2):
            if t < NC:
                c = t
                if c >= 2:
                    x_rdma(c - 2).wait_send()
                ldf = load_f(c)
                ldf.start()
                load_k(c).start()
                if c >= DEPTH:
                    pl.semaphore_wait(credit1_sem, 1)
                ldf.wait()
                sbuf[c % 2] = fbuf[c % 2].astype(jnp.bfloat16)
                x_rdma(c).start()
            if 1 <= t <= NC:
                c = t - 1
                if c >= 2:
                    z_rdma(c - 2).wait_send()
                    store1(c - 2).wait()
                x_rdma(c).wait_recv()
                load_k(c).wait()
                red = kbuf[c % 2] + recv1[c % DEPTH].astype(jnp.float32)
                zbuf[c % 2] = red.astype(jnp.bfloat16)
                if c <= NC - 1 - DEPTH:
                    pl.semaphore_signal(credit1_sem, inc=1, device_id=xp,
                                        device_id_type=pl.DeviceIdType.MESH)
                if c >= DEPTH:
                    pl.semaphore_wait(credit2_sem, 1)
                z_rdma(c).start()
                store1(c).start()
            if t >= 2:
                c = t - 2
                z_rdma(c).wait_recv()
                st2 = store2(c)
                st2.start()
                st2.wait()
                if c <= NC - 1 - DEPTH:
                    pl.semaphore_signal(credit2_sem, inc=1, device_id=zp,
                                        device_id_type=pl.DeviceIdType.MESH)

        for c in (NC - 2, NC - 1):
            x_rdma(c).wait_send()
            z_rdma(c).wait_send()
            store1(c).wait()

    out = pl.pallas_call(
        body,
        out_shape=jax.ShapeDtypeStruct((M, N), jnp.bfloat16),
        in_specs=[pl.BlockSpec(memory_space=pltpu.MemorySpace.HBM)],
        out_specs=pl.BlockSpec(memory_space=pltpu.MemorySpace.HBM),
        scratch_shapes=[
            pltpu.VMEM((2, CH, N), jnp.float32),
            pltpu.VMEM((2, CH, N), jnp.float32),
            pltpu.VMEM((2, CH, N), jnp.bfloat16),
            pltpu.VMEM((2, CH, N), jnp.bfloat16),
            pltpu.VMEM((DEPTH, CH, N), jnp.bfloat16),
            pltpu.VMEM((DEPTH, CH, N), jnp.bfloat16),
            pltpu.SemaphoreType.DMA((2,)),
            pltpu.SemaphoreType.DMA((2,)),
            pltpu.SemaphoreType.DMA((2,)),
            pltpu.SemaphoreType.DMA((2,)),
            pltpu.SemaphoreType.DMA((2,)),
            pltpu.SemaphoreType.DMA((2,)),
            pltpu.SemaphoreType.DMA((DEPTH,)),
            pltpu.SemaphoreType.DMA((DEPTH,)),
            pltpu.SemaphoreType.REGULAR,
            pltpu.SemaphoreType.REGULAR,
        ],
        compiler_params=pltpu.CompilerParams(
            collective_id=0, vmem_limit_bytes=100 * 1024 * 1024),
    )(x)
    return out


# device time: 434278 ns/iter; 2.1444x vs baseline; 1.0023x over previous
import jax
import jax.numpy as jnp
from jax import lax
from jax.experimental import pallas as pl
from jax.experimental.pallas import tpu as pltpu

M = 32768
N = 1024
HALF = M // 2
CH = 1024
NC = HALF // CH
LOOK = 3
DEPTH = 8


def kernel(x):
    def body(x_ref, out_ref,
             fbuf, kbuf, sbuf, zbuf, recv1,
             ldf_sems, ldk_sems, st1_sems,
             send1_sems, send2_sems, recv1_sems, recv2_sems,
             credit1_sem):
        my_x = lax.axis_index("x")
        my_y = lax.axis_index("y")
        my_z = lax.axis_index("z")
        p = my_z % 2
        K = my_x ^ p
        xp = (1 - my_x, my_y, my_z)
        zp = (my_x, my_y, my_z ^ 1)
        k_base = K * HALF
        s_base = (1 - K) * HALF

        def x_rdma(c):
            return pltpu.make_async_remote_copy(
                src_ref=sbuf.at[c % 2],
                dst_ref=recv1.at[c % DEPTH],
                send_sem=send1_sems.at[c % 2],
                recv_sem=recv1_sems.at[c % DEPTH],
                device_id=xp,
                device_id_type=pl.DeviceIdType.MESH,
            )

        def z_rdma(c):
            return pltpu.make_async_remote_copy(
                src_ref=zbuf.at[c % 2],
                dst_ref=out_ref.at[pl.ds(k_base + c * CH, CH)],
                send_sem=send2_sems.at[c % 2],
                recv_sem=recv2_sems.at[c],
                device_id=zp,
                device_id_type=pl.DeviceIdType.MESH,
            )

        def z_recv(c):
            return pltpu.make_async_remote_copy(
                src_ref=zbuf.at[c % 2],
                dst_ref=out_ref.at[pl.ds(s_base + c * CH, CH)],
                send_sem=send2_sems.at[c % 2],
                recv_sem=recv2_sems.at[c],
                device_id=zp,
                device_id_type=pl.DeviceIdType.MESH,
            )

        def load_f(c):
            return pltpu.make_async_copy(
                x_ref.at[pl.ds(s_base + c * CH, CH)],
                fbuf.at[c % LOOK], ldf_sems.at[c % LOOK])

        def load_k(c):
            return pltpu.make_async_copy(
                x_ref.at[pl.ds(k_base + c * CH, CH)],
                kbuf.at[c % LOOK], ldk_sems.at[c % LOOK])

        def store1(c):
            return pltpu.make_async_copy(
                zbuf.at[c % 2],
                out_ref.at[pl.ds(k_base + c * CH, CH)], st1_sems.at[c % 2])

        barrier_sem = pltpu.get_barrier_semaphore()
        for nbr in (xp, zp):
            pl.semaphore_signal(barrier_sem, inc=1, device_id=nbr,
                                device_id_type=pl.DeviceIdType.MESH)
        pl.semaphore_wait(barrier_sem, 2)

        for c in range(LOOK):
            load_f(c).start()
            load_k(c).start()

        for t in range(NC + 2):
            if t < NC:
                c = t
                if c >= 2:
                    x_rdma(c - 2).wait_send()
                load_f(c).wait()
                sbuf[c % 2] = fbuf[c % LOOK].astype(jnp.bfloat16)
                if c >= DEPTH:
                    pl.semaphore_wait(credit1_sem, 1)
                x_rdma(c).start()
                if c + LOOK < NC:
                    load_f(c + LOOK).start()
            if 1 <= t <= NC:
                c = t - 1
                if c >= 2:
                    z_rdma(c - 2).wait_send()
                    store1(c - 2).wait()
                x_rdma(c).wait_recv()
                load_k(c).wait()
                red = kbuf[c % LOOK] + recv1[c % DEPTH].astype(jnp.float32)
                zbuf[c % 2] = red.astype(jnp.bfloat16)
                if c <= NC - 1 - DEPTH:
                    pl.semaphore_signal(credit1_sem, inc=1, device_id=xp,
                                        device_id_type=pl.DeviceIdType.MESH)
                z_rdma(c).start()
                store1(c).start()
                if c + LOOK < NC:
                    load_k(c + LOOK).start()
            if t >= 2:
                z_recv(t - 2).wait_recv()

        for c in (NC - 2, NC - 1):
            x_rdma(c).wait_send()
            z_rdma(c).wait_send()
            store1(c).wait()

    out = pl.pallas_call(
        body,
        out_shape=jax.ShapeDtypeStruct((M, N), jnp.bfloat16),
        in_specs=[pl.BlockSpec(memory_space=pltpu.MemorySpace.HBM)],
        out_specs=pl.BlockSpec(memory_space=pltpu.MemorySpace.HBM),
        scratch_shapes=[
            pltpu.VMEM((LOOK, CH, N), jnp.float32),
            pltpu.VMEM((LOOK, CH, N), jnp.float32),
            pltpu.VMEM((2, CH, N), jnp.bfloat16),
            pltpu.VMEM((2, CH, N), jnp.bfloat16),
            pltpu.VMEM((DEPTH, CH, N), jnp.bfloat16),
            pltpu.SemaphoreType.DMA((LOOK,)),
            pltpu.SemaphoreType.DMA((LOOK,)),
            pltpu.SemaphoreType.DMA((2,)),
            pltpu.SemaphoreType.DMA((2,)),
            pltpu.SemaphoreType.DMA((2,)),
            pltpu.SemaphoreType.DMA((DEPTH,)),
            pltpu.SemaphoreType.DMA((NC,)),
            pltpu.SemaphoreType.REGULAR,
        ],
        compiler_params=pltpu.CompilerParams(
            collective_id=0, vmem_limit_bytes=100 * 1024 * 1024),
    )(x)
    return out
